# Initial kernel scaffold; baseline (speedup 1.0000x reference)
import jax
import jax.numpy as jnp
from jax.experimental import pallas as pl


def kernel(mid_batch_ph, mid_his_batch_ph, mask, mid_embeddings_var, W, b):
    raise NotImplementedError("write your pallas kernel here")


# speedup vs baseline: 1.0985x; 1.0985x over previous
import functools

import jax
import jax.numpy as jnp
from jax import lax
from jax.experimental import pallas as pl
from jax.experimental.pallas import tpu as pltpu
from jax.experimental.pallas import tpu_sc as plsc

N_MID = 1000000
EMB = 64
HID = 64
B = 4096
SEQ = 200

NC = 2
NS = 16
NW = NC * NS
BPW = B // NW
CHUNK = 100
CPE = SEQ // CHUNK
NCHUNK = BPW * CPE
NBUF = 4
L = 16


def _sc_body(table, hist_idx, item_idx, out_sum, out_item,
             idx_v, buf_v, pooled_v, iidx_v, item_v, sems, isem):
  wid = lax.axis_index("s") * NC + lax.axis_index("c")

  pltpu.sync_copy(hist_idx.at[wid], idx_v)
  pltpu.sync_copy(item_idx.at[wid], iidx_v)

  pltpu.make_async_copy(table.at[iidx_v], item_v, isem).start()

  def fire(c, b):
    pltpu.make_async_copy(table.at[idx_v.at[c]], buf_v.at[b], sems.at[b]).start()

  for b in range(NBUF):
    fire(b, b)

  def accum_chunk(b, accs):
    def s_body(s, accs):
      a0, a1, a2, a3 = accs
      a0 = a0 + buf_v[b, s, pl.ds(0, L)]
      a1 = a1 + buf_v[b, s, pl.ds(L, L)]
      a2 = a2 + buf_v[b, s, pl.ds(2 * L, L)]
      a3 = a3 + buf_v[b, s, pl.ds(3 * L, L)]
      return (a0, a1, a2, a3)
    return lax.fori_loop(0, CHUNK, s_body, accs, unroll=4)

  zeros = jnp.zeros((L,), jnp.float32)

  def outer(i, carry):
    c0 = i * NBUF
    for e in range(NBUF // CPE):
      accs = (zeros, zeros, zeros, zeros)
      for h in range(CPE):
        b = e * CPE + h
        c = c0 + b
        pltpu.make_async_copy(
            table.at[idx_v.at[c]], buf_v.at[b], sems.at[b]).wait()
        accs = accum_chunk(b, accs)

        @pl.when(c + NBUF < NCHUNK)
        def _():
          fire(c + NBUF, b)
      row = i * (NBUF // CPE) + e
      for j in range(EMB // L):
        pooled_v[row, pl.ds(j * L, L)] = accs[j]
    return carry

  lax.fori_loop(0, NCHUNK // NBUF, outer, 0)

  pltpu.sync_copy(pooled_v, out_sum.at[pl.ds(wid * BPW, BPW)])
  pltpu.make_async_copy(table.at[iidx_v], item_v, isem).wait()
  pltpu.sync_copy(item_v, out_item.at[pl.ds(wid * BPW, BPW)])


@functools.partial(jax.jit, static_argnames=())
def _sc_gather(table, hist_idx, item_idx):
  mesh = plsc.VectorSubcoreMesh(core_axis_name="c", subcore_axis_name="s")
  kern = pl.kernel(
      _sc_body,
      out_type=(
          jax.ShapeDtypeStruct((B, EMB), jnp.float32),
          jax.ShapeDtypeStruct((B, EMB), jnp.float32),
      ),
      mesh=mesh,
      scratch_types=[
          pltpu.VMEM((NCHUNK, CHUNK), jnp.int32),
          pltpu.VMEM((NBUF, CHUNK, EMB), jnp.float32),
          pltpu.VMEM((BPW, EMB), jnp.float32),
          pltpu.VMEM((BPW,), jnp.int32),
          pltpu.VMEM((BPW, EMB), jnp.float32),
          pltpu.SemaphoreType.DMA((NBUF,)),
          pltpu.SemaphoreType.DMA,
      ],
      compiler_params=pltpu.CompilerParams(use_tc_tiling_on_sc=False),
      name="sc_embedding_bag",
  )
  return kern(table, hist_idx, item_idx)


def _mm_body(sum_ref, mask_ref, w_ref, b_ref, out_ref):
  den = jnp.sum(mask_ref[...], axis=1, keepdims=True) + 1e-9
  mean = sum_ref[...] / den
  out_ref[...] = (
      jnp.dot(mean, w_ref[...], preferred_element_type=jnp.float32)
      + b_ref[...]
  )


def _project(pooled_sum, mask, W, b):
  return pl.pallas_call(
      _mm_body,
      out_shape=jax.ShapeDtypeStruct((B, HID), jnp.float32),
      name="mean_dense",
  )(pooled_sum, mask, W, b.reshape(1, HID))


def kernel(mid_batch_ph, mid_his_batch_ph, mask, mid_embeddings_var, W, b):
  hist_idx = mid_his_batch_ph.reshape(NW, NCHUNK, CHUNK)
  item_idx = mid_batch_ph.reshape(NW, BPW)
  pooled_sum, item_eb = _sc_gather(mid_embeddings_var, hist_idx, item_idx)
  user_eb = _project(pooled_sum, mask, W, b)
  return (user_eb, item_eb)
